# baseline (speedup 1.0000x reference)
import functools

import jax
import jax.numpy as jnp
from jax import lax
from jax.experimental import pallas as pl
from jax.experimental.pallas import tpu as pltpu
from jax.experimental.pallas import tpu_sc as plsc

BS = 7
SC_PLANES = 128
NWORKERS = 32
L = 16


def _rotl_ops(x1, r):
    return lax.shift_left(x1, jnp.uint32(r)) | lax.shift_right_logical(
        x1, jnp.uint32(32 - r))


def _threefry_bits(idx):
    rotations = ((13, 15, 26, 6), (17, 29, 16, 24))
    ks = (jnp.uint32(0), jnp.uint32(42), jnp.uint32(42 ^ 0x1BD11BDA))
    x0 = jnp.zeros_like(idx) + ks[0]
    x1 = idx + ks[1]
    for i in range(5):
        for r in rotations[i % 2]:
            x0 = x0 + x1
            x1 = _rotl_ops(x1, r)
            x1 = x0 ^ x1
        x0 = x0 + ks[(i + 1) % 3]
        x1 = x1 + ks[(i + 2) % 3] + jnp.uint32(i + 1)
    return x0 ^ x1


def _win7_max(p, axis, out_len):
    def sl(a, start, length):
        idx = [slice(None)] * a.ndim
        idx[axis] = slice(start, start + length)
        return a[tuple(idx)]

    n = p.shape[axis]
    s1 = jnp.maximum(sl(p, 0, n - 1), sl(p, 1, n - 1))
    s2 = jnp.maximum(sl(s1, 0, n - 3), sl(s1, 2, n - 3))
    return jnp.maximum(sl(s2, 0, out_len), sl(s2, 3, out_len))


def _mask_kernel(gamma_ref, base_ref, cmask_ref, mask_ref, count_ref, *,
                 G, mh, mw, H, W):
    step = pl.program_id(0)
    g = gamma_ref[0]

    idx = base_ref[...] + (step * (G * mh * mw)).astype(jnp.uint32)
    bits = _threefry_bits(idx)

    mant = lax.shift_right_logical(bits, jnp.uint32(9))
    thresh = jnp.ceil(g * jnp.float32(8388608.0)).astype(jnp.uint32)
    sel = jnp.where(mant < thresh, jnp.float32(1.0), jnp.float32(0.0))
    mask = sel.astype(jnp.bfloat16) * cmask_ref[...]

    zr = jnp.zeros((G, BS - 1, W), jnp.bfloat16)
    zb = jnp.zeros((G, H - mh, W), jnp.bfloat16)
    pr = jnp.concatenate([zr, mask, zb], axis=1)
    rm = _win7_max(pr, 1, H)
    zc = jnp.zeros((G, H, BS - 1), jnp.bfloat16)
    pc = jnp.concatenate([zc, rm], axis=2)
    dl = _win7_max(pc, 2, W)

    bm = jnp.bfloat16(1.0) - dl
    mask_ref[...] = bm.astype(jnp.int8)

    lane = lax.broadcasted_iota(jnp.int32, (1, 128), 1)
    count_ref[0, ...] = jnp.where(lane == 0,
                                  jnp.sum(bm.astype(jnp.float32)),
                                  jnp.float32(0.0))


def _sc_mask_kernel(gamma_hbm, mask_hbm, cnt_hbm, bufa, bufr, gamma_v, cnt_v,
                    *, tc_planes, mh, mw, H, W, ppw):
    nchunks = W // L
    wid = lax.axis_index("s") * 2 + lax.axis_index("c")

    pltpu.sync_copy(gamma_hbm, gamma_v)
    thresh = gamma_v[...][0] * jnp.float32(8388608.0)

    lane_i = lax.iota(jnp.int32, L)
    lane_u = lane_i.astype(jnp.uint32)
    zero_v = jnp.zeros((L,), jnp.float32)
    tailmask = lane_i < jnp.int32(mw - (W - L))

    cnt_v[...] = zero_v
    for pr_ in range(H, H + BS - 1):
        for j in range(nchunks):
            bufa[pr_, pl.ds(j * L, L)] = zero_v
    def _plane_body(pi, _):
        plane = wid * ppw + pi
        pbase = (jnp.int32(tc_planes) + plane) * jnp.int32(mh * mw)

        def _zr_body(r, _):
            for j in range(nchunks):
                bufa[r, pl.ds(j * L, L)] = zero_v
            return 0
        lax.fori_loop(0, BS - 1, _zr_body, 0)

        def _prng_body(r, _):
            rbase = pbase + r * jnp.int32(mw)
            for j in range(nchunks):
                idx = lane_u + (rbase + j * L).astype(jnp.uint32)
                bits = _threefry_bits(idx)
                mant = lax.shift_right_logical(bits, jnp.uint32(9))
                m = jnp.where(mant.astype(jnp.float32) < thresh,
                              jnp.float32(1.0), jnp.float32(0.0))
                if j == nchunks - 1:
                    m = jnp.where(tailmask, m, jnp.float32(0.0))
                bufa[(BS - 1) + r, pl.ds(j * L, L)] = m
            return 0
        lax.fori_loop(0, mh, _prng_body, 0)

        def _rowd_body(p, _):
            for j in range(nchunks):
                v = bufa[p, pl.ds(j * L, L)]
                for k in range(1, BS):
                    v = jnp.maximum(v, bufa[p + k, pl.ds(j * L, L)])
                bufr[p, pl.ds(j * L, L)] = v
            return 0
        lax.fori_loop(0, H, _rowd_body, 0)

        def _cold_body(p, _):
            p_v = jnp.zeros((L,), jnp.int32) + p
            for j in range(nchunks):
                v = bufr[p, pl.ds(j * L, L)]
                col = lane_i + jnp.int32(j * L)
                for k in range(1, BS):
                    if j == 0:
                        src = jnp.maximum(col - jnp.int32(k), jnp.int32(0))
                        g = plsc.load_gather(bufr, [p_v, src])
                        g = jnp.where(col >= jnp.int32(k), g,
                                      jnp.float32(0.0))
                    else:
                        g = plsc.load_gather(bufr, [p_v, col - jnp.int32(k)])
                    v = jnp.maximum(v, g)
                bm = jnp.float32(1.0) - v
                bufa[p, pl.ds(j * L, L)] = bm
                cnt_v[...] += bm
            return 0
        lax.fori_loop(0, H, _cold_body, 0)

        pltpu.sync_copy(bufa.at[pl.ds(0, H)], mask_hbm.at[plane])
        return 0

    lax.fori_loop(0, ppw, _plane_body, 0)
    pltpu.sync_copy(cnt_v, cnt_hbm.at[wid])


def _scale_kernel(tcc_ref, scc_ref, x_ref, mi8_ref, mf32_ref, o_ref, *,
                  count_m, split):
    step = pl.program_id(0)
    scale = jnp.float32(count_m) / (jnp.sum(tcc_ref[...])
                                    + jnp.sum(scc_ref[...]))

    @pl.when(step < split)
    def _tc_part():
        o_ref[...] = x_ref[...] * mi8_ref[...].astype(jnp.float32) * scale

    @pl.when(step >= split)
    def _sc_part():
        o_ref[...] = x_ref[...] * mf32_ref[...] * scale


def kernel(x, gamma):
    B, C, H, W = x.shape
    mh, mw = H - (BS - 1), W - (BS - 1)
    nplanes = B * C
    count_m = nplanes * H * W
    tc_planes = nplanes - SC_PLANES
    ppw = SC_PLANES // NWORKERS

    gamma16 = jnp.broadcast_to(gamma, (L,))
    sc_mask, sc_counts = pl.kernel(
        functools.partial(_sc_mask_kernel, tc_planes=tc_planes, mh=mh, mw=mw,
                          H=H, W=W, ppw=ppw),
        out_type=[
            jax.ShapeDtypeStruct((SC_PLANES, H, W), jnp.float32),
            jax.ShapeDtypeStruct((NWORKERS, L), jnp.float32),
        ],
        mesh=plsc.VectorSubcoreMesh(core_axis_name="c", subcore_axis_name="s"),
        scratch_types=[
            pltpu.VMEM((H + BS - 1, W), jnp.float32),
            pltpu.VMEM((H, W), jnp.float32),
            pltpu.VMEM((L,), jnp.float32),
            pltpu.VMEM((L,), jnp.float32),
        ],
        compiler_params=pltpu.CompilerParams(needs_layout_passes=False),
    )(gamma16)

    G = 16
    nsteps = tc_planes // G
    base = (jnp.arange(G, dtype=jnp.uint32)[:, None, None] * (mh * mw)
            + jnp.arange(mh, dtype=jnp.uint32)[:, None] * mw
            + jnp.arange(W, dtype=jnp.uint32)[None, :])
    cmask = (jnp.arange(W) < mw).astype(jnp.bfloat16)[None, None, :]

    mask_i8, tc_counts = pl.pallas_call(
        functools.partial(_mask_kernel, G=G, mh=mh, mw=mw, H=H, W=W),
        grid=(nsteps,),
        in_specs=[
            pl.BlockSpec(memory_space=pltpu.SMEM),
            pl.BlockSpec((G, mh, W), lambda i: (0, 0, 0)),
            pl.BlockSpec((1, 1, W), lambda i: (0, 0, 0)),
        ],
        out_specs=[
            pl.BlockSpec((G, H, W), lambda i: (i, 0, 0)),
            pl.BlockSpec((1, 1, 128), lambda i: (i, 0, 0)),
        ],
        out_shape=[
            jax.ShapeDtypeStruct((tc_planes, H, W), jnp.int8),
            jax.ShapeDtypeStruct((nsteps, 1, 128), jnp.float32),
        ],
        compiler_params=pltpu.CompilerParams(
            dimension_semantics=("parallel",)),
    )(gamma, base, cmask)

    G2 = 32
    split = tc_planes // G2
    x3 = x.reshape(nplanes, H, W)
    out = pl.pallas_call(
        functools.partial(_scale_kernel, count_m=count_m, split=split),
        grid=(nplanes // G2,),
        in_specs=[
            pl.BlockSpec((nsteps, 1, 128), lambda i: (0, 0, 0)),
            pl.BlockSpec((NWORKERS, L), lambda i: (0, 0)),
            pl.BlockSpec((G2, H, W), lambda i: (i, 0, 0)),
            pl.BlockSpec((G2, H, W),
                         lambda i: (jnp.minimum(i, split - 1), 0, 0)),
            pl.BlockSpec((G2, H, W),
                         lambda i: (jnp.maximum(i - split, 0), 0, 0)),
        ],
        out_specs=pl.BlockSpec((G2, H, W), lambda i: (i, 0, 0)),
        out_shape=jax.ShapeDtypeStruct((nplanes, H, W), jnp.float32),
        compiler_params=pltpu.CompilerParams(
            dimension_semantics=("parallel",)),
    )(tc_counts, sc_counts, x3, mask_i8, sc_mask)

    return out.reshape(B, C, H, W)


# speedup vs baseline: 1.0404x; 1.0404x over previous
import functools

import jax
import jax.numpy as jnp
from jax import lax
from jax.experimental import pallas as pl
from jax.experimental.pallas import tpu as pltpu
from jax.experimental.pallas import tpu_sc as plsc

BS = 7
SC_PLANES = 160
NWORKERS = 32
L = 16


def _rotl_ops(x1, r):
    return lax.shift_left(x1, jnp.uint32(r)) | lax.shift_right_logical(
        x1, jnp.uint32(32 - r))


def _threefry_bits(idx):
    rotations = ((13, 15, 26, 6), (17, 29, 16, 24))
    ks = (jnp.uint32(0), jnp.uint32(42), jnp.uint32(42 ^ 0x1BD11BDA))
    x0 = jnp.zeros_like(idx) + ks[0]
    x1 = idx + ks[1]
    for i in range(5):
        for r in rotations[i % 2]:
            x0 = x0 + x1
            x1 = _rotl_ops(x1, r)
            x1 = x0 ^ x1
        x0 = x0 + ks[(i + 1) % 3]
        x1 = x1 + ks[(i + 2) % 3] + jnp.uint32(i + 1)
    return x0 ^ x1


def _win7_max(p, axis, out_len):
    def sl(a, start, length):
        idx = [slice(None)] * a.ndim
        idx[axis] = slice(start, start + length)
        return a[tuple(idx)]

    n = p.shape[axis]
    s1 = jnp.maximum(sl(p, 0, n - 1), sl(p, 1, n - 1))
    s2 = jnp.maximum(sl(s1, 0, n - 3), sl(s1, 2, n - 3))
    return jnp.maximum(sl(s2, 0, out_len), sl(s2, 3, out_len))


def _mask_kernel(gamma_ref, base_ref, cmask_ref, mask_ref, count_ref, *,
                 G, mh, mw, H, W):
    step = pl.program_id(0)
    g = gamma_ref[0]

    idx = base_ref[...] + (step * (G * mh * mw)).astype(jnp.uint32)
    bits = _threefry_bits(idx)

    mant = lax.shift_right_logical(bits, jnp.uint32(9))
    thresh = jnp.ceil(g * jnp.float32(8388608.0)).astype(jnp.uint32)
    sel = jnp.where(mant < thresh, jnp.float32(1.0), jnp.float32(0.0))
    mask = sel.astype(jnp.bfloat16) * cmask_ref[...]

    zr = jnp.zeros((G, BS - 1, W), jnp.bfloat16)
    zb = jnp.zeros((G, H - mh, W), jnp.bfloat16)
    pr = jnp.concatenate([zr, mask, zb], axis=1)
    rm = _win7_max(pr, 1, H)
    zc = jnp.zeros((G, H, BS - 1), jnp.bfloat16)
    pc = jnp.concatenate([zc, rm], axis=2)
    dl = _win7_max(pc, 2, W)

    bm = jnp.bfloat16(1.0) - dl
    mask_ref[...] = bm.astype(jnp.int8)

    lane = lax.broadcasted_iota(jnp.int32, (1, 128), 1)
    count_ref[0, ...] = jnp.where(lane == 0,
                                  jnp.sum(bm.astype(jnp.float32)),
                                  jnp.float32(0.0))


def _sc_mask_kernel(gamma_hbm, mask_hbm, cnt_hbm, bufa, bufr, gamma_v, cnt_v,
                    *, tc_planes, mh, mw, H, W, ppw):
    nchunks = W // L
    wid = lax.axis_index("s") * 2 + lax.axis_index("c")

    pltpu.sync_copy(gamma_hbm, gamma_v)
    thresh = gamma_v[...][0] * jnp.float32(8388608.0)

    lane_i = lax.iota(jnp.int32, L)
    lane_u = lane_i.astype(jnp.uint32)
    zero_v = jnp.zeros((L,), jnp.float32)
    tailmask = lane_i < jnp.int32(mw - (W - L))

    cnt_v[...] = zero_v
    for pr_ in range(H, H + BS - 1):
        for j in range(nchunks):
            bufa[pr_, pl.ds(j * L, L)] = zero_v
    def _plane_body(pi, cnt):
        plane = wid * ppw + pi
        pbase = (jnp.int32(tc_planes) + plane) * jnp.int32(mh * mw)

        @plsc.parallel_loop(0, BS - 1)
        def _zr_body(r):
            for j in range(nchunks):
                bufa[r, pl.ds(j * L, L)] = zero_v

        @plsc.parallel_loop(0, mh)
        def _prng_body(r):
            rbase = pbase + r * jnp.int32(mw)
            for j in range(nchunks):
                idx = lane_u + (rbase + j * L).astype(jnp.uint32)
                bits = _threefry_bits(idx)
                mant = lax.shift_right_logical(bits, jnp.uint32(9))
                m = jnp.where(mant.astype(jnp.float32) < thresh,
                              jnp.float32(1.0), jnp.float32(0.0))
                if j == nchunks - 1:
                    m = jnp.where(tailmask, m, jnp.float32(0.0))
                bufa[(BS - 1) + r, pl.ds(j * L, L)] = m

        @plsc.parallel_loop(0, H)
        def _rowd_body(p):
            for j in range(nchunks):
                v = bufa[p, pl.ds(j * L, L)]
                for k in range(1, BS):
                    v = jnp.maximum(v, bufa[p + k, pl.ds(j * L, L)])
                bufr[p, pl.ds(j * L, L)] = v

        def _cold_body(p, c):
            p_v = jnp.zeros((L,), jnp.int32) + p
            for j in range(nchunks):
                v = bufr[p, pl.ds(j * L, L)]
                col = lane_i + jnp.int32(j * L)
                for k in range(1, BS):
                    if j == 0:
                        src = jnp.maximum(col - jnp.int32(k), jnp.int32(0))
                        g = plsc.load_gather(bufr, [p_v, src])
                        g = jnp.where(col >= jnp.int32(k), g,
                                      jnp.float32(0.0))
                    else:
                        g = plsc.load_gather(bufr, [p_v, col - jnp.int32(k)])
                    v = jnp.maximum(v, g)
                bm = jnp.float32(1.0) - v
                bufa[p, pl.ds(j * L, L)] = bm
                c = c + bm
            return c
        cnt = plsc.parallel_loop(0, H, carry=cnt)(_cold_body)

        pltpu.sync_copy(bufa.at[pl.ds(0, H)], mask_hbm.at[plane])
        return cnt

    cnt = lax.fori_loop(0, ppw, _plane_body, zero_v)
    cnt_v[...] = cnt
    pltpu.sync_copy(cnt_v, cnt_hbm.at[wid])


def _scale_kernel(tcc_ref, scc_ref, x_ref, mi8_ref, mf32_ref, o_ref, *,
                  count_m, split):
    step = pl.program_id(0)
    scale = jnp.float32(count_m) / (jnp.sum(tcc_ref[...])
                                    + jnp.sum(scc_ref[...]))

    @pl.when(step < split)
    def _tc_part():
        o_ref[...] = x_ref[...] * mi8_ref[...].astype(jnp.float32) * scale

    @pl.when(step >= split)
    def _sc_part():
        o_ref[...] = x_ref[...] * mf32_ref[...] * scale


def kernel(x, gamma):
    B, C, H, W = x.shape
    mh, mw = H - (BS - 1), W - (BS - 1)
    nplanes = B * C
    count_m = nplanes * H * W
    tc_planes = nplanes - SC_PLANES
    ppw = SC_PLANES // NWORKERS

    gamma16 = jnp.broadcast_to(gamma, (L,))
    sc_mask, sc_counts = pl.kernel(
        functools.partial(_sc_mask_kernel, tc_planes=tc_planes, mh=mh, mw=mw,
                          H=H, W=W, ppw=ppw),
        out_type=[
            jax.ShapeDtypeStruct((SC_PLANES, H, W), jnp.float32),
            jax.ShapeDtypeStruct((NWORKERS, L), jnp.float32),
        ],
        mesh=plsc.VectorSubcoreMesh(core_axis_name="c", subcore_axis_name="s"),
        scratch_types=[
            pltpu.VMEM((H + BS - 1, W), jnp.float32),
            pltpu.VMEM((H, W), jnp.float32),
            pltpu.VMEM((L,), jnp.float32),
            pltpu.VMEM((L,), jnp.float32),
        ],
        compiler_params=pltpu.CompilerParams(needs_layout_passes=False),
    )(gamma16)

    G = 16
    nsteps = tc_planes // G
    base = (jnp.arange(G, dtype=jnp.uint32)[:, None, None] * (mh * mw)
            + jnp.arange(mh, dtype=jnp.uint32)[:, None] * mw
            + jnp.arange(W, dtype=jnp.uint32)[None, :])
    cmask = (jnp.arange(W) < mw).astype(jnp.bfloat16)[None, None, :]

    mask_i8, tc_counts = pl.pallas_call(
        functools.partial(_mask_kernel, G=G, mh=mh, mw=mw, H=H, W=W),
        grid=(nsteps,),
        in_specs=[
            pl.BlockSpec(memory_space=pltpu.SMEM),
            pl.BlockSpec((G, mh, W), lambda i: (0, 0, 0)),
            pl.BlockSpec((1, 1, W), lambda i: (0, 0, 0)),
        ],
        out_specs=[
            pl.BlockSpec((G, H, W), lambda i: (i, 0, 0)),
            pl.BlockSpec((1, 1, 128), lambda i: (i, 0, 0)),
        ],
        out_shape=[
            jax.ShapeDtypeStruct((tc_planes, H, W), jnp.int8),
            jax.ShapeDtypeStruct((nsteps, 1, 128), jnp.float32),
        ],
        compiler_params=pltpu.CompilerParams(
            dimension_semantics=("parallel",)),
    )(gamma, base, cmask)

    G2 = 32
    split = tc_planes // G2
    x3 = x.reshape(nplanes, H, W)
    out = pl.pallas_call(
        functools.partial(_scale_kernel, count_m=count_m, split=split),
        grid=(nplanes // G2,),
        in_specs=[
            pl.BlockSpec((nsteps, 1, 128), lambda i: (0, 0, 0)),
            pl.BlockSpec((NWORKERS, L), lambda i: (0, 0)),
            pl.BlockSpec((G2, H, W), lambda i: (i, 0, 0)),
            pl.BlockSpec((G2, H, W),
                         lambda i: (jnp.minimum(i, split - 1), 0, 0)),
            pl.BlockSpec((G2, H, W),
                         lambda i: (jnp.maximum(i - split, 0), 0, 0)),
        ],
        out_specs=pl.BlockSpec((G2, H, W), lambda i: (i, 0, 0)),
        out_shape=jax.ShapeDtypeStruct((nplanes, H, W), jnp.float32),
        compiler_params=pltpu.CompilerParams(
            dimension_semantics=("parallel",)),
    )(tc_counts, sc_counts, x3, mask_i8, sc_mask)

    return out.reshape(B, C, H, W)


# speedup vs baseline: 1.0431x; 1.0027x over previous
import functools

import jax
import jax.numpy as jnp
from jax import lax
from jax.experimental import pallas as pl
from jax.experimental.pallas import tpu as pltpu
from jax.experimental.pallas import tpu_sc as plsc

BS = 7
SC_PLANES = 192
NWORKERS = 32
L = 16


def _rotl_ops(x1, r):
    return lax.shift_left(x1, jnp.uint32(r)) | lax.shift_right_logical(
        x1, jnp.uint32(32 - r))


def _threefry_bits(idx):
    rotations = ((13, 15, 26, 6), (17, 29, 16, 24))
    ks = (jnp.uint32(0), jnp.uint32(42), jnp.uint32(42 ^ 0x1BD11BDA))
    x0 = jnp.zeros_like(idx) + ks[0]
    x1 = idx + ks[1]
    for i in range(5):
        for r in rotations[i % 2]:
            x0 = x0 + x1
            x1 = _rotl_ops(x1, r)
            x1 = x0 ^ x1
        x0 = x0 + ks[(i + 1) % 3]
        x1 = x1 + ks[(i + 2) % 3] + jnp.uint32(i + 1)
    return x0 ^ x1


def _win7_max(p, axis, out_len):
    def sl(a, start, length):
        idx = [slice(None)] * a.ndim
        idx[axis] = slice(start, start + length)
        return a[tuple(idx)]

    n = p.shape[axis]
    s1 = jnp.maximum(sl(p, 0, n - 1), sl(p, 1, n - 1))
    s2 = jnp.maximum(sl(s1, 0, n - 3), sl(s1, 2, n - 3))
    return jnp.maximum(sl(s2, 0, out_len), sl(s2, 3, out_len))


def _mask_kernel(gamma_ref, base_ref, cmask_ref, mask_ref, count_ref, *,
                 G, mh, mw, H, W):
    step = pl.program_id(0)
    g = gamma_ref[0]

    idx = base_ref[...] + (step * (G * mh * mw)).astype(jnp.uint32)
    bits = _threefry_bits(idx)

    mant = lax.shift_right_logical(bits, jnp.uint32(9))
    thresh = jnp.ceil(g * jnp.float32(8388608.0)).astype(jnp.uint32)
    sel = jnp.where(mant < thresh, jnp.float32(1.0), jnp.float32(0.0))
    mask = sel.astype(jnp.bfloat16) * cmask_ref[...]

    zr = jnp.zeros((G, BS - 1, W), jnp.bfloat16)
    zb = jnp.zeros((G, H - mh, W), jnp.bfloat16)
    pr = jnp.concatenate([zr, mask, zb], axis=1)
    rm = _win7_max(pr, 1, H)
    zc = jnp.zeros((G, H, BS - 1), jnp.bfloat16)
    pc = jnp.concatenate([zc, rm], axis=2)
    dl = _win7_max(pc, 2, W)

    bm = jnp.bfloat16(1.0) - dl
    mask_ref[...] = bm.astype(jnp.int8)

    lane = lax.broadcasted_iota(jnp.int32, (1, 128), 1)
    count_ref[0, ...] = jnp.where(lane == 0,
                                  jnp.sum(bm.astype(jnp.float32)),
                                  jnp.float32(0.0))


def _sc_mask_kernel(gamma_hbm, mask_hbm, cnt_hbm, bufa, bufr, gamma_v, cnt_v,
                    *, tc_planes, mh, mw, H, W, ppw):
    nchunks = W // L
    wid = lax.axis_index("s") * 2 + lax.axis_index("c")

    pltpu.sync_copy(gamma_hbm, gamma_v)
    thresh = gamma_v[...][0] * jnp.float32(8388608.0)

    lane_i = lax.iota(jnp.int32, L)
    lane_u = lane_i.astype(jnp.uint32)
    zero_v = jnp.zeros((L,), jnp.float32)
    tailmask = lane_i < jnp.int32(mw - (W - L))

    cnt_v[...] = zero_v
    for pr_ in range(H, H + BS - 1):
        for j in range(nchunks):
            bufa[pr_, pl.ds(j * L, L)] = zero_v
    def _plane_body(pi, cnt):
        plane = wid * ppw + pi
        pbase = (jnp.int32(tc_planes) + plane) * jnp.int32(mh * mw)

        @plsc.parallel_loop(0, BS - 1)
        def _zr_body(r):
            for j in range(nchunks):
                bufa[r, pl.ds(j * L, L)] = zero_v

        @plsc.parallel_loop(0, mh)
        def _prng_body(r):
            rbase = pbase + r * jnp.int32(mw)
            for j in range(nchunks):
                idx = lane_u + (rbase + j * L).astype(jnp.uint32)
                bits = _threefry_bits(idx)
                mant = lax.shift_right_logical(bits, jnp.uint32(9))
                m = jnp.where(mant.astype(jnp.float32) < thresh,
                              jnp.float32(1.0), jnp.float32(0.0))
                if j == nchunks - 1:
                    m = jnp.where(tailmask, m, jnp.float32(0.0))
                bufa[(BS - 1) + r, pl.ds(j * L, L)] = m

        @plsc.parallel_loop(0, H)
        def _rowd_body(p):
            for j in range(nchunks):
                v = bufa[p, pl.ds(j * L, L)]
                for k in range(1, BS):
                    v = jnp.maximum(v, bufa[p + k, pl.ds(j * L, L)])
                bufr[p, pl.ds(j * L, L)] = v

        def _cold_body(p, c):
            p_v = jnp.zeros((L,), jnp.int32) + p
            for j in range(nchunks):
                v = bufr[p, pl.ds(j * L, L)]
                col = lane_i + jnp.int32(j * L)
                for k in range(1, BS):
                    if j == 0:
                        src = jnp.maximum(col - jnp.int32(k), jnp.int32(0))
                        g = plsc.load_gather(bufr, [p_v, src])
                        g = jnp.where(col >= jnp.int32(k), g,
                                      jnp.float32(0.0))
                    else:
                        g = plsc.load_gather(bufr, [p_v, col - jnp.int32(k)])
                    v = jnp.maximum(v, g)
                bm = jnp.float32(1.0) - v
                bufa[p, pl.ds(j * L, L)] = bm
                c = c + bm
            return c
        cnt = plsc.parallel_loop(0, H, carry=cnt)(_cold_body)

        pltpu.sync_copy(bufa.at[pl.ds(0, H)], mask_hbm.at[plane])
        return cnt

    cnt = lax.fori_loop(0, ppw, _plane_body, zero_v)
    cnt_v[...] = cnt
    pltpu.sync_copy(cnt_v, cnt_hbm.at[wid])


def _scale_kernel(tcc_ref, scc_ref, x_ref, mi8_ref, mf32_ref, o_ref, *,
                  count_m, split):
    step = pl.program_id(0)
    scale = jnp.float32(count_m) / (jnp.sum(tcc_ref[...])
                                    + jnp.sum(scc_ref[...]))

    @pl.when(step < split)
    def _tc_part():
        o_ref[...] = x_ref[...] * mi8_ref[...].astype(jnp.float32) * scale

    @pl.when(step >= split)
    def _sc_part():
        o_ref[...] = x_ref[...] * mf32_ref[...] * scale


def kernel(x, gamma):
    B, C, H, W = x.shape
    mh, mw = H - (BS - 1), W - (BS - 1)
    nplanes = B * C
    count_m = nplanes * H * W
    tc_planes = nplanes - SC_PLANES
    ppw = SC_PLANES // NWORKERS

    gamma16 = jnp.broadcast_to(gamma, (L,))
    sc_mask, sc_counts = pl.kernel(
        functools.partial(_sc_mask_kernel, tc_planes=tc_planes, mh=mh, mw=mw,
                          H=H, W=W, ppw=ppw),
        out_type=[
            jax.ShapeDtypeStruct((SC_PLANES, H, W), jnp.float32),
            jax.ShapeDtypeStruct((NWORKERS, L), jnp.float32),
        ],
        mesh=plsc.VectorSubcoreMesh(core_axis_name="c", subcore_axis_name="s"),
        scratch_types=[
            pltpu.VMEM((H + BS - 1, W), jnp.float32),
            pltpu.VMEM((H, W), jnp.float32),
            pltpu.VMEM((L,), jnp.float32),
            pltpu.VMEM((L,), jnp.float32),
        ],
        compiler_params=pltpu.CompilerParams(needs_layout_passes=False),
    )(gamma16)

    G = 16
    nsteps = tc_planes // G
    base = (jnp.arange(G, dtype=jnp.uint32)[:, None, None] * (mh * mw)
            + jnp.arange(mh, dtype=jnp.uint32)[:, None] * mw
            + jnp.arange(W, dtype=jnp.uint32)[None, :])
    cmask = (jnp.arange(W) < mw).astype(jnp.bfloat16)[None, None, :]

    mask_i8, tc_counts = pl.pallas_call(
        functools.partial(_mask_kernel, G=G, mh=mh, mw=mw, H=H, W=W),
        grid=(nsteps,),
        in_specs=[
            pl.BlockSpec(memory_space=pltpu.SMEM),
            pl.BlockSpec((G, mh, W), lambda i: (0, 0, 0)),
            pl.BlockSpec((1, 1, W), lambda i: (0, 0, 0)),
        ],
        out_specs=[
            pl.BlockSpec((G, H, W), lambda i: (i, 0, 0)),
            pl.BlockSpec((1, 1, 128), lambda i: (i, 0, 0)),
        ],
        out_shape=[
            jax.ShapeDtypeStruct((tc_planes, H, W), jnp.int8),
            jax.ShapeDtypeStruct((nsteps, 1, 128), jnp.float32),
        ],
        compiler_params=pltpu.CompilerParams(
            dimension_semantics=("parallel",)),
    )(gamma, base, cmask)

    G2 = 32
    split = tc_planes // G2
    x3 = x.reshape(nplanes, H, W)
    out = pl.pallas_call(
        functools.partial(_scale_kernel, count_m=count_m, split=split),
        grid=(nplanes // G2,),
        in_specs=[
            pl.BlockSpec((nsteps, 1, 128), lambda i: (0, 0, 0)),
            pl.BlockSpec((NWORKERS, L), lambda i: (0, 0)),
            pl.BlockSpec((G2, H, W), lambda i: (i, 0, 0)),
            pl.BlockSpec((G2, H, W),
                         lambda i: (jnp.minimum(i, split - 1), 0, 0)),
            pl.BlockSpec((G2, H, W),
                         lambda i: (jnp.maximum(i - split, 0), 0, 0)),
        ],
        out_specs=pl.BlockSpec((G2, H, W), lambda i: (i, 0, 0)),
        out_shape=jax.ShapeDtypeStruct((nplanes, H, W), jnp.float32),
        compiler_params=pltpu.CompilerParams(
            dimension_semantics=("parallel",)),
    )(tc_counts, sc_counts, x3, mask_i8, sc_mask)

    return out.reshape(B, C, H, W)
